# baseline (speedup 1.0000x reference)
import functools

import jax
import jax.numpy as jnp
from jax import lax
from jax.experimental import pallas as pl
from jax.experimental.pallas import tpu as pltpu
from jax.experimental.pallas import tpu_sc as plsc

G = 16
PK = 8
DTP = 16


def _gelu(t):
    return 0.5 * t * (1.0 + lax.erf(t * 0.7071067811865476))


def _fold(v, times, op):
    for _ in range(times):
        half = v.shape[1] // 2
        v = op(v[:, :half], v[:, half:])
    return v


def _bn_packed(h, gamma_t, beta_t):
    n_nodes = h.shape[0] * PK
    df = h.shape[1] // PK
    folds = PK.bit_length() - 1
    s = _fold(jnp.sum(h, axis=0, keepdims=True), folds, jnp.add)
    mu = jnp.concatenate([s / n_nodes] * PK, axis=1)
    d = h - mu
    sv = _fold(jnp.sum(d * d, axis=0, keepdims=True), folds, jnp.add)
    var = jnp.concatenate([sv / n_nodes] * PK, axis=1)
    return gamma_t * d * lax.rsqrt(var + 1e-5) + beta_t


def _segmax_packed(b128, zp):
    folds = PK.bit_length() - 1
    rows = []
    for g in range(G):
        m = jnp.max(jnp.where(b128 == g, zp, -jnp.inf), axis=0, keepdims=True)
        rows.append(_fold(m, folds, jnp.maximum))
    return jnp.concatenate(rows, axis=0)


def _tc0_body(xp_ref, b128_ref, W0_ref, b0_ref, g0_ref, be0_ref, L0W_ref,
              L0b_ref, x1_ref, z0_ref, o0_ref):
    h = jnp.dot(xp_ref[...], W0_ref[...],
                preferred_element_type=jnp.float32) + b0_ref[...]
    x1 = _gelu(_bn_packed(h, g0_ref[...], be0_ref[...]))
    x1_ref[...] = x1
    z0 = _gelu(jnp.dot(x1, L0W_ref[...],
                       preferred_element_type=jnp.float32) + L0b_ref[...])
    z0_ref[...] = z0
    o0_ref[...] = _segmax_packed(b128_ref[...], z0)


def _tc_gin_body(parts_ref, b128_ref, Wc_ref, bc_ref, g_ref, be_ref,
                 LW_ref, Lb_ref, Zin_ref, oin_ref, xo_ref, Zo_ref, oo_ref):
    agg = parts_ref[0] + parts_ref[1]
    h = jnp.dot(agg, Wc_ref[...],
                preferred_element_type=jnp.float32) + bc_ref[...]
    xn = _bn_packed(h, g_ref[...], be_ref[...])
    xo_ref[...] = xn
    z = jnp.dot(xn, LW_ref[...],
                preferred_element_type=jnp.float32) + Lb_ref[...]
    Zo_ref[...] = Zin_ref[...] + z
    oo_ref[...] = oin_ref[...] + _segmax_packed(b128_ref[...], z)


def _blk(W):
    a, b = W.shape
    eye = jnp.eye(PK, dtype=W.dtype)
    return (eye[:, None, :, None] * W[None, :, None, :]).reshape(PK * a,
                                                                 PK * b)


def _tile(v):
    return jnp.tile(v, PK).reshape(1, -1)


@functools.lru_cache(maxsize=None)
def _make_sc_agg(n, d, e):
    info = plsc.get_sparse_core_info()
    nc, ns = info.num_cores, info.num_subcores
    nw = nc * ns
    epw = e // nw
    ch = 2000
    n_ch = epw // ch
    rpt = (-(-n // ns) + 7) // 8 * 8
    rlast = n - (ns - 1) * rpt
    assert rlast > 0
    mesh = plsc.VectorSubcoreMesh(core_axis_name="c", subcore_axis_name="s")

    @functools.partial(
        pl.kernel,
        out_type=jax.ShapeDtypeStruct((nc, n, d), jnp.float32),
        mesh=mesh,
        compiler_params=pltpu.CompilerParams(use_tc_tiling_on_sc=False),
        scratch_types=[
            [pltpu.VMEM((ch,), jnp.int32)] * 2,
            [pltpu.VMEM((ch,), jnp.int32)] * 2,
            [pltpu.VMEM((ch, d), jnp.float32)] * 2,
            pltpu.VMEM_SHARED((n, d), jnp.float32),
            [pltpu.SemaphoreType.DMA] * 2,
            [pltpu.SemaphoreType.DMA] * 2,
            [pltpu.SemaphoreType.DMA] * 2,
            [pltpu.SemaphoreType.DMA] * 2,
        ],
    )
    def sc_agg(xtab, zeros, src, dst, out, src_b, dst_b, rows_b, acc,
               sem_s, sem_d, sem_g, sem_w):
        c = lax.axis_index("c")
        s = lax.axis_index("s")
        w = s * nc + c
        r0 = s * rpt

        def _seed(off, sz):
            @pl.when(c == 0)
            def _():
                pltpu.async_copy(xtab.at[pl.ds(off, sz)],
                                 acc.at[pl.ds(off, sz)], sem_w[1]).wait()

            @pl.when(c != 0)
            def _():
                pltpu.async_copy(zeros.at[pl.ds(off, sz)],
                                 acc.at[pl.ds(off, sz)], sem_w[1]).wait()

        def _sl(ref, i):
            return ref.at[pl.ds(w * epw + i * ch, ch)]

        def _start_idx(i):
            b = i % 2
            return (pltpu.async_copy(_sl(src, i), src_b[b], sem_s[b]),
                    pltpu.async_copy(_sl(dst, i), dst_b[b], sem_d[b]))

        def _start_gather(i):
            b = i % 2
            return pltpu.async_copy(xtab.at[src_b[b]], rows_b[b], sem_g[b])

        hi0 = _start_idx(0)
        hi = _start_idx(1)

        @pl.when(s < ns - 1)
        def _():
            _seed(r0, rpt)

        @pl.when(s == ns - 1)
        def _():
            _seed((ns - 1) * rpt, rlast)

        plsc.subcore_barrier()

        hi0[0].wait()
        hi0[1].wait()
        hg = _start_gather(0)
        for i in range(n_ch):
            hg_next = None
            if i + 1 < n_ch:
                hi[0].wait()
                hi[1].wait()
                hg_next = _start_gather(i + 1)
            hg.wait()
            b = i % 2
            pltpu.sync_copy(rows_b[b], acc.at[dst_b[b]], add=True)
            if i + 2 < n_ch:
                hi = _start_idx(i + 2)
            hg = hg_next

        plsc.subcore_barrier()

        @pl.when(s < ns - 1)
        def _():
            pltpu.sync_copy(acc.at[pl.ds(r0, rpt)], out.at[c, pl.ds(r0, rpt)])

        @pl.when(s == ns - 1)
        def _():
            off = (ns - 1) * rpt
            pltpu.sync_copy(acc.at[pl.ds(off, rlast)],
                            out.at[c, pl.ds(off, rlast)])

    return sc_agg


def kernel(x, edge_index, batch, W0, b0, g0, be0, L0W, L0b, Wc1, bc1, g1,
           be1, L1W, L1b, Wc2, bc2, g2, be2, L2W, L2b):
    n, dfin = x.shape
    e = edge_index.shape[1]
    dt = L0W.shape[1]
    dh = W0.shape[1]
    d2 = Wc2.shape[1]
    rp = n // PK

    src = edge_index[0]
    dst = edge_index[1]
    zeros_tab = jnp.zeros((n, dh), jnp.float32)

    xp = x.reshape(rp, PK * dfin)
    b128 = jnp.repeat(batch.reshape(rp, PK), DTP, axis=1)
    padt = lambda W: jnp.pad(W, ((0, 0), (0, DTP - dt)))
    padb = lambda v: jnp.pad(v, (0, DTP - dt))
    f32 = jnp.float32

    tc0 = pl.pallas_call(
        _tc0_body,
        out_shape=[
            jax.ShapeDtypeStruct((rp, PK * dh), f32),
            jax.ShapeDtypeStruct((rp, PK * DTP), f32),
            jax.ShapeDtypeStruct((G, DTP), f32),
        ],
    )
    x1p, z0p, o0 = tc0(xp, b128, _blk(W0), _tile(b0), _tile(g0), _tile(be0),
                       _blk(padt(L0W)), _tile(padb(L0b)))

    sc_agg = _make_sc_agg(n, dh, e)
    parts1 = sc_agg(x1p.reshape(n, dh), zeros_tab, src, dst)

    tc_gin16 = pl.pallas_call(
        _tc_gin_body,
        out_shape=[
            jax.ShapeDtypeStruct((rp, PK * dh), f32),
            jax.ShapeDtypeStruct((rp, PK * DTP), f32),
            jax.ShapeDtypeStruct((G, DTP), f32),
        ],
    )
    x2p, Z1p, o1 = tc_gin16(parts1.reshape(2, rp, PK * dh), b128, _blk(Wc1),
                            _tile(bc1), _tile(g1), _tile(be1),
                            _blk(padt(L1W)), _tile(padb(L1b)), z0p, o0)

    parts2 = sc_agg(x2p.reshape(n, dh), zeros_tab, src, dst)

    tc_gin8 = pl.pallas_call(
        _tc_gin_body,
        out_shape=[
            jax.ShapeDtypeStruct((rp, PK * d2), f32),
            jax.ShapeDtypeStruct((rp, PK * DTP), f32),
            jax.ShapeDtypeStruct((G, DTP), f32),
        ],
    )
    x3p, Zp, o2 = tc_gin8(parts2.reshape(2, rp, PK * dh), b128, _blk(Wc2),
                          _tile(bc2), _tile(g2), _tile(be2),
                          _blk(padt(L2W)), _tile(padb(L2b)), Z1p, o1)

    out = o2[:, :dt]
    Z = Zp.reshape(n, DTP)[:, :dt]
    x3 = x3p.reshape(n, d2)
    return (out, Z, x3)


# speedup vs baseline: 1.0452x; 1.0452x over previous
import functools

import jax
import jax.numpy as jnp
from jax import lax
from jax.experimental import pallas as pl
from jax.experimental.pallas import tpu as pltpu
from jax.experimental.pallas import tpu_sc as plsc

G = 16
PK = 8
DTP = 16


def _gelu(t):
    return 0.5 * t * (1.0 + lax.erf(t * 0.7071067811865476))


def _fold(v, times, op):
    for _ in range(times):
        half = v.shape[1] // 2
        v = op(v[:, :half], v[:, half:])
    return v


def _bn_packed(h, gamma_t, beta_t):
    n_nodes = h.shape[0] * PK
    df = h.shape[1] // PK
    folds = PK.bit_length() - 1
    s = _fold(jnp.sum(h, axis=0, keepdims=True), folds, jnp.add)
    mu = jnp.concatenate([s / n_nodes] * PK, axis=1)
    d = h - mu
    sv = _fold(jnp.sum(d * d, axis=0, keepdims=True), folds, jnp.add)
    var = jnp.concatenate([sv / n_nodes] * PK, axis=1)
    return gamma_t * d * lax.rsqrt(var + 1e-5) + beta_t


def _segmax_packed(b128, zp):
    folds = PK.bit_length() - 1
    rows = []
    for g in range(G):
        m = jnp.max(jnp.where(b128 == g, zp, -jnp.inf), axis=0, keepdims=True)
        rows.append(_fold(m, folds, jnp.maximum))
    return jnp.concatenate(rows, axis=0)


def _tc0_body(xp_ref, b128_ref, W0_ref, b0_ref, g0_ref, be0_ref, L0W_ref,
              L0b_ref, x1_ref, z0_ref, o0_ref):
    h = jnp.dot(xp_ref[...], W0_ref[...],
                preferred_element_type=jnp.float32) + b0_ref[...]
    x1 = _gelu(_bn_packed(h, g0_ref[...], be0_ref[...]))
    x1_ref[...] = x1
    z0 = _gelu(jnp.dot(x1, L0W_ref[...],
                       preferred_element_type=jnp.float32) + L0b_ref[...])
    z0_ref[...] = z0
    o0_ref[...] = _segmax_packed(b128_ref[...], z0)


def _tc_gin_body(parts_ref, b128_ref, Wc_ref, bc_ref, g_ref, be_ref,
                 LW_ref, Lb_ref, Zin_ref, oin_ref, xo_ref, Zo_ref, oo_ref):
    agg = parts_ref[0] + parts_ref[1]
    h = jnp.dot(agg, Wc_ref[...],
                preferred_element_type=jnp.float32) + bc_ref[...]
    xn = _bn_packed(h, g_ref[...], be_ref[...])
    xo_ref[...] = xn
    z = jnp.dot(xn, LW_ref[...],
                preferred_element_type=jnp.float32) + Lb_ref[...]
    Zo_ref[...] = Zin_ref[...] + z
    oo_ref[...] = oin_ref[...] + _segmax_packed(b128_ref[...], z)


def _blk(W):
    a, b = W.shape
    out = jnp.zeros((PK, a, PK, b), W.dtype)
    for i in range(PK):
        out = out.at[i, :, i, :].set(W)
    return out.reshape(PK * a, PK * b)


def _tile(v):
    return jnp.tile(v, PK).reshape(1, -1)


@functools.lru_cache(maxsize=None)
def _make_sc_agg(n, d, e):
    info = plsc.get_sparse_core_info()
    nc, ns = info.num_cores, info.num_subcores
    nw = nc * ns
    epw = e // nw
    ch = 2000
    n_ch = epw // ch
    rpt = (-(-n // ns) + 7) // 8 * 8
    rlast = n - (ns - 1) * rpt
    assert rlast > 0
    mesh = plsc.VectorSubcoreMesh(core_axis_name="c", subcore_axis_name="s")

    @functools.partial(
        pl.kernel,
        out_type=jax.ShapeDtypeStruct((nc, n, d), jnp.float32),
        mesh=mesh,
        compiler_params=pltpu.CompilerParams(use_tc_tiling_on_sc=False),
        scratch_types=[
            [pltpu.VMEM((ch,), jnp.int32)] * 2,
            [pltpu.VMEM((ch,), jnp.int32)] * 2,
            [pltpu.VMEM((ch, d), jnp.float32)] * 2,
            pltpu.VMEM_SHARED((n, d), jnp.float32),
            pltpu.VMEM_SHARED((n, d), jnp.float32),
            [pltpu.SemaphoreType.DMA] * 2,
            [pltpu.SemaphoreType.DMA] * 2,
            [pltpu.SemaphoreType.DMA] * 2,
        ],
    )
    def sc_agg(xtab, zeros, src, dst, out, src_b, dst_b, rows_b, xs, acc,
               sem_s, sem_d, sem_g):
        c = lax.axis_index("c")
        s = lax.axis_index("s")
        w = s * nc + c
        r0 = s * rpt

        def _seed(off, sz):
            pltpu.sync_copy(xtab.at[pl.ds(off, sz)], xs.at[pl.ds(off, sz)])

            @pl.when(c == 0)
            def _():
                pltpu.sync_copy(xtab.at[pl.ds(off, sz)], acc.at[pl.ds(off, sz)])

            @pl.when(c != 0)
            def _():
                pltpu.sync_copy(zeros.at[pl.ds(off, sz)], acc.at[pl.ds(off, sz)])

        @pl.when(s < ns - 1)
        def _():
            _seed(r0, rpt)

        @pl.when(s == ns - 1)
        def _():
            _seed((ns - 1) * rpt, rlast)

        plsc.subcore_barrier()

        def _sl(ref, i):
            return ref.at[pl.ds(w * epw + i * ch, ch)]

        def _start_idx(i):
            b = i % 2
            return (pltpu.async_copy(_sl(src, i), src_b[b], sem_s[b]),
                    pltpu.async_copy(_sl(dst, i), dst_b[b], sem_d[b]))

        def _start_gather(i):
            b = i % 2
            return pltpu.async_copy(xs.at[src_b[b]], rows_b[b], sem_g[b])

        hi = _start_idx(0)
        hi[0].wait()
        hi[1].wait()
        hg = _start_gather(0)
        hi = _start_idx(1)
        for i in range(n_ch):
            hg_next = None
            if i + 1 < n_ch:
                hi[0].wait()
                hi[1].wait()
                hg_next = _start_gather(i + 1)
            hg.wait()
            b = i % 2
            pltpu.sync_copy(rows_b[b], acc.at[dst_b[b]], add=True)
            if i + 2 < n_ch:
                hi = _start_idx(i + 2)
            hg = hg_next

        plsc.subcore_barrier()

        @pl.when(s < ns - 1)
        def _():
            pltpu.sync_copy(acc.at[pl.ds(r0, rpt)], out.at[c, pl.ds(r0, rpt)])

        @pl.when(s == ns - 1)
        def _():
            off = (ns - 1) * rpt
            pltpu.sync_copy(acc.at[pl.ds(off, rlast)],
                            out.at[c, pl.ds(off, rlast)])

    return sc_agg


def kernel(x, edge_index, batch, W0, b0, g0, be0, L0W, L0b, Wc1, bc1, g1,
           be1, L1W, L1b, Wc2, bc2, g2, be2, L2W, L2b):
    n, dfin = x.shape
    e = edge_index.shape[1]
    dt = L0W.shape[1]
    dh = W0.shape[1]
    d2 = Wc2.shape[1]
    rp = n // PK

    src = edge_index[0]
    dst = edge_index[1]
    zeros_tab = jnp.zeros((n, dh), jnp.float32)

    xp = x.reshape(rp, PK * dfin)
    b128 = jnp.repeat(batch.reshape(rp, PK), DTP, axis=1)
    padt = lambda W: jnp.pad(W, ((0, 0), (0, DTP - dt)))
    padb = lambda v: jnp.pad(v, (0, DTP - dt))
    f32 = jnp.float32

    tc0 = pl.pallas_call(
        _tc0_body,
        out_shape=[
            jax.ShapeDtypeStruct((rp, PK * dh), f32),
            jax.ShapeDtypeStruct((rp, PK * DTP), f32),
            jax.ShapeDtypeStruct((G, DTP), f32),
        ],
    )
    x1p, z0p, o0 = tc0(xp, b128, _blk(W0), _tile(b0), _tile(g0), _tile(be0),
                       _blk(padt(L0W)), _tile(padb(L0b)))

    sc_agg = _make_sc_agg(n, dh, e)
    parts1 = sc_agg(x1p.reshape(n, dh), zeros_tab, src, dst)

    tc_gin16 = pl.pallas_call(
        _tc_gin_body,
        out_shape=[
            jax.ShapeDtypeStruct((rp, PK * dh), f32),
            jax.ShapeDtypeStruct((rp, PK * DTP), f32),
            jax.ShapeDtypeStruct((G, DTP), f32),
        ],
    )
    x2p, Z1p, o1 = tc_gin16(parts1.reshape(2, rp, PK * dh), b128, _blk(Wc1),
                            _tile(bc1), _tile(g1), _tile(be1),
                            _blk(padt(L1W)), _tile(padb(L1b)), z0p, o0)

    parts2 = sc_agg(x2p.reshape(n, dh), zeros_tab, src, dst)

    tc_gin8 = pl.pallas_call(
        _tc_gin_body,
        out_shape=[
            jax.ShapeDtypeStruct((rp, PK * d2), f32),
            jax.ShapeDtypeStruct((rp, PK * DTP), f32),
            jax.ShapeDtypeStruct((G, DTP), f32),
        ],
    )
    x3p, Zp, o2 = tc_gin8(parts2.reshape(2, rp, PK * dh), b128, _blk(Wc2),
                          _tile(bc2), _tile(g2), _tile(be2),
                          _blk(padt(L2W)), _tile(padb(L2b)), Z1p, o1)

    out = o2[:, :dt]
    Z = Zp.reshape(n, DTP)[:, :dt]
    x3 = x3p.reshape(n, d2)
    return (out, Z, x3)
